# Initial kernel scaffold; baseline (speedup 1.0000x reference)
import jax
import jax.numpy as jnp
from jax.experimental import pallas as pl


def kernel(cla_map, reg_map, anchor):
    raise NotImplementedError("write your pallas kernel here")


# speedup vs baseline: 2.7116x; 2.7116x over previous
import functools

import jax
import jax.numpy as jnp
import numpy as np
from jax import lax
from jax.experimental import pallas as pl
from jax.experimental.pallas import tpu as pltpu
from jax.experimental.pallas import tpu_sc as plsc

_SRC = 600.0
_LOGIT = float(np.log(0.7) - np.log(0.3))

_A, _H, _W = 9, 48, 48
_PLANE = _H * _W
_L = 16
_JB = 4
_ROWS = _H // _JB
_UE = _ROWS * _W
_NU = _A * _JB
_NW = 32


def _run_unit(u, cla, reg, anchor, out,
              c0, c1, t0, t1, t2, t3, idxv, aux, obuf, isem, gsem):
    p = u // _JB
    jb = u % _JB
    iota = lax.iota(jnp.int32, _L)

    for c in range(4):
        for g in range(3):
            idxv[pl.ds(c * _W + g * _L, _L)] = (
                (iota + (g * _L)) * (6 * (_W + 1) * p) + (2 + c))
    gh = pltpu.async_copy(anchor.at[idxv], aux, gsem)

    cbase = (2 * p) * _PLANE + jb * _UE
    rbase = (4 * p) * _PLANE + jb * _UE
    hs = [
        pltpu.async_copy(cla.at[pl.ds(cbase, _UE)], c0, isem),
        pltpu.async_copy(cla.at[pl.ds(cbase + _PLANE, _UE)], c1, isem),
        pltpu.async_copy(reg.at[pl.ds(rbase, _UE)], t0, isem),
        pltpu.async_copy(reg.at[pl.ds(rbase + _PLANE, _UE)], t1, isem),
        pltpu.async_copy(reg.at[pl.ds(rbase + 2 * _PLANE, _UE)], t2, isem),
        pltpu.async_copy(reg.at[pl.ds(rbase + 3 * _PLANE, _UE)], t3, isem),
    ]
    for h in hs:
        h.wait()
    gh.wait()

    for g in range(3):
        acx = aux[pl.ds(0 * _W + g * _L, _L)]
        acy = aux[pl.ds(1 * _W + g * _L, _L)]
        aw = aux[pl.ds(2 * _W + g * _L, _L)]
        ah = aux[pl.ds(3 * _W + g * _L, _L)]
        for j in range(_ROWS):
            off = j * _W + g * _L
            sl = pl.ds(off, _L)
            c0v = c0[sl]
            c1v = c1[sl]
            cx = (t0[sl] * aw + acx) * _SRC
            cy = (t1[sl] * ah + acy) * _SRC
            wv = jnp.exp(t2[sl]) * aw * _SRC
            hv = jnp.exp(t3[sl]) * ah * _SRC
            wh = wv * 0.5
            hh = hv * 0.5
            ltx = cx - wh
            lty = cy - hh
            rbx = cx + wh
            rby = cy + hh
            m = ((c1v - c0v > _LOGIT)
                 & (ltx >= 0.0) & (lty >= 0.0)
                 & (rbx <= _SRC) & (rby <= _SRC))
            vals = (ltx, lty, rbx, rby,
                    cx * (1.0 / _SRC), cy * (1.0 / _SRC),
                    wv * (1.0 / _SRC), hv * (1.0 / _SRC))
            for c, v in enumerate(vals):
                plsc.store_scatter(obuf, [(iota + off) * 8 + c],
                                   jnp.where(m, v, 0.0))

    pltpu.sync_copy(obuf, out.at[pl.ds((p * _PLANE + jb * _UE) * 8, _UE * 8)])


def kernel(cla_map, reg_map, anchor):
    cla = cla_map.reshape(-1)
    reg = reg_map.reshape(-1)
    anc = anchor.reshape(-1)
    mesh = plsc.VectorSubcoreMesh(core_axis_name="c", subcore_axis_name="s",
                                  num_cores=2, num_subcores=16)

    @functools.partial(
        pl.kernel,
        out_type=jax.ShapeDtypeStruct((_A * _PLANE * 8,), jnp.float32),
        mesh=mesh,
        compiler_params=pltpu.CompilerParams(needs_layout_passes=False),
        scratch_types=[
            pltpu.VMEM((_UE,), jnp.float32),
            pltpu.VMEM((_UE,), jnp.float32),
            pltpu.VMEM((_UE,), jnp.float32),
            pltpu.VMEM((_UE,), jnp.float32),
            pltpu.VMEM((_UE,), jnp.float32),
            pltpu.VMEM((_UE,), jnp.float32),
            pltpu.VMEM((4 * _W,), jnp.int32),
            pltpu.VMEM((4 * _W,), jnp.float32),
            pltpu.VMEM((_UE * 8,), jnp.float32),
            pltpu.SemaphoreType.DMA,
            pltpu.SemaphoreType.DMA,
        ],
    )
    def sc_kernel(cla_h, reg_h, anc_h, out_h,
                  c0, c1, t0, t1, t2, t3, idxv, aux, obuf, isem, gsem):
        wid = lax.axis_index("s") * 2 + lax.axis_index("c")
        args = (cla_h, reg_h, anc_h, out_h,
                c0, c1, t0, t1, t2, t3, idxv, aux, obuf, isem, gsem)
        _run_unit(wid, *args)

        @pl.when(wid < _NU - _NW)
        def _():
            _run_unit(wid + _NW, *args)

    return sc_kernel(cla, reg, anc).reshape(_A * _PLANE, 8)


# speedup vs baseline: 3.7958x; 1.3998x over previous
import functools

import jax
import jax.numpy as jnp
import numpy as np
from jax import lax
from jax.experimental import pallas as pl
from jax.experimental.pallas import tpu as pltpu
from jax.experimental.pallas import tpu_sc as plsc

_SRC = 600.0
_LOGIT = float(np.log(0.7) - np.log(0.3))

_A, _H, _W = 9, 48, 48
_L = 16
_JB = 3
_ROWS = _H // _JB
_UE = _ROWS * _W
_NU = _A * _JB


def kernel(cla_map, reg_map, anchor):
    mesh = plsc.VectorSubcoreMesh(core_axis_name="c", subcore_axis_name="s",
                                  num_cores=2, num_subcores=16)

    @functools.partial(
        pl.kernel,
        out_type=jax.ShapeDtypeStruct((_A * _H * _W, 8), jnp.float32),
        mesh=mesh,
        compiler_params=pltpu.CompilerParams(needs_layout_passes=False),
        scratch_types=[
            pltpu.VMEM((_ROWS, _W), jnp.float32),
            pltpu.VMEM((_ROWS, _W), jnp.float32),
            pltpu.VMEM((_ROWS, _W), jnp.float32),
            pltpu.VMEM((_ROWS, _W), jnp.float32),
            pltpu.VMEM((_ROWS, _W), jnp.float32),
            pltpu.VMEM((_ROWS, _W), jnp.float32),
            pltpu.VMEM((4 * _W,), jnp.int32),
            pltpu.VMEM((4 * _W,), jnp.float32),
            pltpu.VMEM((_UE, 8), jnp.float32),
            pltpu.SemaphoreType.DMA,
            pltpu.SemaphoreType.DMA,
        ],
    )
    def sc_kernel(cla, reg, anc, out,
                  c0, c1, t0, t1, t2, t3, idxv, aux, obuf, isem, gsem):
        wid = lax.axis_index("s") * 2 + lax.axis_index("c")

        @pl.when(wid < _NU)
        def _():
            p = wid // _JB
            j0 = (wid % _JB) * _ROWS
            iota = lax.iota(jnp.int32, _L)

            for c in range(4):
                for g in range(3):
                    idxv[pl.ds(c * _W + g * _L, _L)] = (
                        (iota + (g * _L)) * (4 * p) + c)
            gh = pltpu.async_copy(anc.at[idxv], aux, gsem)

            hs = [
                pltpu.async_copy(cla.at[0, 2 * p, pl.ds(j0, _ROWS)], c0, isem),
                pltpu.async_copy(cla.at[0, 2 * p + 1, pl.ds(j0, _ROWS)], c1, isem),
                pltpu.async_copy(reg.at[0, 4 * p, pl.ds(j0, _ROWS)], t0, isem),
                pltpu.async_copy(reg.at[0, 4 * p + 1, pl.ds(j0, _ROWS)], t1, isem),
                pltpu.async_copy(reg.at[0, 4 * p + 2, pl.ds(j0, _ROWS)], t2, isem),
                pltpu.async_copy(reg.at[0, 4 * p + 3, pl.ds(j0, _ROWS)], t3, isem),
            ]
            for h in hs:
                h.wait()
            gh.wait()

            for g in range(3):
                acx = aux[pl.ds(0 * _W + g * _L, _L)]
                acy = aux[pl.ds(1 * _W + g * _L, _L)]
                aw = aux[pl.ds(2 * _W + g * _L, _L)]
                ah = aux[pl.ds(3 * _W + g * _L, _L)]
                for j in range(_ROWS):
                    sl = pl.ds(g * _L, _L)
                    c0v = c0[j, sl]
                    c1v = c1[j, sl]
                    cx = (t0[j, sl] * aw + acx) * _SRC
                    cy = (t1[j, sl] * ah + acy) * _SRC
                    wv = jnp.exp(t2[j, sl]) * aw * _SRC
                    hv = jnp.exp(t3[j, sl]) * ah * _SRC
                    wh = wv * 0.5
                    hh = hv * 0.5
                    ltx = cx - wh
                    lty = cy - hh
                    rbx = cx + wh
                    rby = cy + hh
                    m = ((c1v - c0v > _LOGIT)
                         & (ltx >= 0.0) & (lty >= 0.0)
                         & (rbx <= _SRC) & (rby <= _SRC))
                    rows = iota + (j * _W + g * _L)
                    vals = (ltx, lty, rbx, rby,
                            cx * (1.0 / _SRC), cy * (1.0 / _SRC),
                            wv * (1.0 / _SRC), hv * (1.0 / _SRC))
                    for c, v in enumerate(vals):
                        plsc.store_scatter(
                            obuf, [rows, jnp.full((_L,), c, jnp.int32)],
                            jnp.where(m, v, 0.0))

            pltpu.sync_copy(obuf, out.at[pl.ds((p * _H + j0) * _W, _UE)])

    anc = lax.slice(anchor, (0, 2), (49 * 376 + 1, 6), (49, 1)).reshape(-1)
    return sc_kernel(cla_map, reg_map, anc)
